# baseline (speedup 1.0000x reference)
import functools

import jax
import jax.numpy as jnp
from jax import lax
from jax.experimental import pallas as pl
from jax.experimental.pallas import tpu as pltpu
from jax.experimental.pallas import tpu_sc as plsc

B = 16384
K = 21
DIM = 10
V = 1000000

NW = 32
SPT = B // NW
CS = 128
NCHUNK = SPT // CS


def _build_kernel():
    mesh = plsc.VectorSubcoreMesh(core_axis_name="c", subcore_axis_name="s")

    @functools.partial(
        pl.kernel,
        mesh=mesh,
        compiler_params=pltpu.CompilerParams(needs_layout_passes=False,
                                             use_tc_tiling_on_sc=False),
        out_type=jax.ShapeDtypeStruct((K, B), jnp.float32),
        scratch_types=[
            pltpu.VMEM((NCHUNK, CS), jnp.int32),
            pltpu.VMEM((DIM * NCHUNK, CS), jnp.float32),
            pltpu.VMEM((K, CS), jnp.int32),
            pltpu.VMEM((DIM * K, CS), jnp.float32),
            pltpu.VMEM((K, SPT), jnp.float32),
            pltpu.SemaphoreType.DMA,
            pltpu.SemaphoreType.DMA,
        ],
    )
    def sc_kernel(*refs):
        idx_in_hbm = refs[0]
        oidx_refs = refs[1:1 + K]
        win8, win2, wout8, wout2 = refs[1 + K:5 + K]
        out_hbm = refs[5 + K]
        (iidx_v, in_vals_v, oidx_v, vals_v, out_v,
         sem_in, sem_out) = refs[6 + K:]
        win_refs = [win8.at[d] if d < 8 else win2.at[d - 8]
                    for d in range(DIM)]
        wout_refs = [wout8.at[d] if d < 8 else wout2.at[d - 8]
                     for d in range(DIM)]

        wid = lax.axis_index("c") * 16 + lax.axis_index("s")
        base = wid * SPT

        for p in range(NCHUNK):
            pltpu.sync_copy(idx_in_hbm.at[pl.ds(base + p * CS, CS)],
                            iidx_v.at[p])

        for d in range(DIM):
            def fire_in(p, carry, d=d):
                pltpu.async_copy(win_refs[d].at[iidx_v.at[p]],
                                 in_vals_v.at[d * NCHUNK + p], sem_in)
                return carry

            lax.fori_loop(0, NCHUNK, fire_in, 0)

        def drain_in():
            for d in range(DIM):
                def drain1(p, carry, d=d):
                    pltpu.make_async_copy(win_refs[d].at[iidx_v.at[p]],
                                          in_vals_v.at[d * NCHUNK + p],
                                          sem_in).wait()
                    return carry

                lax.fori_loop(0, NCHUNK, drain1, 0)

        def chunk_body(c, carry):
            for k in range(K):
                pltpu.sync_copy(oidx_refs[k].at[pl.ds(base + c * CS, CS)],
                                oidx_v.at[k])

            for d in range(DIM):
                def fire_out(k, carry2, d=d):
                    pltpu.async_copy(wout_refs[d].at[oidx_v.at[k]],
                                     vals_v.at[d * K + k], sem_out)
                    return carry2

                lax.fori_loop(0, K, fire_out, 0)

            @pl.when(c == 0)
            def _():
                drain_in()

            for d in range(DIM):
                def drain_out(k, carry2, d=d):
                    pltpu.make_async_copy(wout_refs[d].at[oidx_v.at[k]],
                                          vals_v.at[d * K + k],
                                          sem_out).wait()
                    return carry2

                lax.fori_loop(0, K, drain_out, 0)

            def grp(g, carry2):
                sbase = g * 16
                ivs = [in_vals_v[d * NCHUNK + c, pl.ds(sbase, 16)]
                       for d in range(DIM)]
                for k in range(K):
                    acc = vals_v[k, pl.ds(sbase, 16)] * ivs[0]
                    for d in range(1, DIM):
                        acc = acc + vals_v[d * K + k, pl.ds(sbase, 16)] * ivs[d]
                    out_v[k, pl.ds(c * CS + sbase, 16)] = acc
                return carry2

            lax.fori_loop(0, CS // 16, grp, 0)
            return carry

        lax.fori_loop(0, NCHUNK, chunk_body, 0)

        for k in range(K):
            pltpu.sync_copy(out_v.at[k], out_hbm.at[k, pl.ds(base, SPT)])

    return sc_kernel


_SC_KERNEL = _build_kernel()


@jax.jit
def kernel(input_index_batch, output_indices_batch, W_in, W_out):
    iidx = input_index_batch.astype(jnp.int32).reshape(B)
    oidx = output_indices_batch.astype(jnp.int32)
    oidx_cols = [oidx[:, k] for k in range(K)]
    woutT = W_out.T
    outT = _SC_KERNEL(iidx, *oidx_cols,
                      W_in[0:8], W_in[8:10], woutT[0:8], woutT[8:10])
    return outT.T


# speedup vs baseline: 5.5914x; 5.5914x over previous
import functools

import jax
import jax.numpy as jnp
from jax import lax
from jax.experimental import pallas as pl
from jax.experimental.pallas import tpu as pltpu
from jax.experimental.pallas import tpu_sc as plsc

B = 16384
K = 21
DIM = 10
V = 1000000

NW = 32
SPT = B // NW
CS = 128
NCHUNK = SPT // CS


def _build_kernel():
    mesh = plsc.VectorSubcoreMesh(core_axis_name="c", subcore_axis_name="s")

    @functools.partial(
        pl.kernel,
        mesh=mesh,
        compiler_params=pltpu.CompilerParams(needs_layout_passes=False,
                                             use_tc_tiling_on_sc=False),
        out_type=jax.ShapeDtypeStruct((K, B), jnp.float32),
        scratch_types=[
            pltpu.VMEM((NCHUNK, CS), jnp.int32),
            pltpu.VMEM((DIM * NCHUNK, CS), jnp.float32),
            pltpu.VMEM((K, CS), jnp.int32),
            pltpu.VMEM((DIM * K, CS), jnp.float32),
            pltpu.VMEM((K, SPT), jnp.float32),
            pltpu.SemaphoreType.DMA,
            pltpu.SemaphoreType.DMA,
        ],
    )
    def sc_kernel(*refs):
        idx_in_hbm = refs[0]
        oidx_refs = refs[1:1 + K]
        win_refs = refs[1 + K:1 + K + DIM]
        wout_refs = refs[1 + K + DIM:1 + K + 2 * DIM]
        out_hbm = refs[1 + K + 2 * DIM]
        (iidx_v, in_vals_v, oidx_v, vals_v, out_v,
         sem_in, sem_out) = refs[2 + K + 2 * DIM:]

        wid = lax.axis_index("c") * 16 + lax.axis_index("s")
        base = wid * SPT

        for p in range(NCHUNK):
            pltpu.sync_copy(idx_in_hbm.at[pl.ds(base + p * CS, CS)],
                            iidx_v.at[p])

        for d in range(DIM):
            def fire_in(p, carry, d=d):
                pltpu.async_copy(win_refs[d].at[iidx_v.at[p]],
                                 in_vals_v.at[d * NCHUNK + p], sem_in)
                return carry

            lax.fori_loop(0, NCHUNK, fire_in, 0)

        def drain_in():
            for d in range(DIM):
                def drain1(p, carry, d=d):
                    pltpu.make_async_copy(win_refs[d].at[iidx_v.at[p]],
                                          in_vals_v.at[d * NCHUNK + p],
                                          sem_in).wait()
                    return carry

                lax.fori_loop(0, NCHUNK, drain1, 0)

        def chunk_body(c, carry):
            for k in range(K):
                pltpu.sync_copy(oidx_refs[k].at[pl.ds(base + c * CS, CS)],
                                oidx_v.at[k])

            for d in range(DIM):
                def fire_out(k, carry2, d=d):
                    pltpu.async_copy(wout_refs[d].at[oidx_v.at[k]],
                                     vals_v.at[d * K + k], sem_out)
                    return carry2

                lax.fori_loop(0, K, fire_out, 0)

            @pl.when(c == 0)
            def _():
                drain_in()

            for d in range(DIM):
                def drain_out(k, carry2, d=d):
                    pltpu.make_async_copy(wout_refs[d].at[oidx_v.at[k]],
                                          vals_v.at[d * K + k],
                                          sem_out).wait()
                    return carry2

                lax.fori_loop(0, K, drain_out, 0)

            def grp(g, carry2):
                sbase = g * 16
                ivs = [in_vals_v[d * NCHUNK + c, pl.ds(sbase, 16)]
                       for d in range(DIM)]
                for k in range(K):
                    acc = vals_v[k, pl.ds(sbase, 16)] * ivs[0]
                    for d in range(1, DIM):
                        acc = acc + vals_v[d * K + k, pl.ds(sbase, 16)] * ivs[d]
                    out_v[k, pl.ds(c * CS + sbase, 16)] = acc
                return carry2

            lax.fori_loop(0, CS // 16, grp, 0)
            return carry

        lax.fori_loop(0, NCHUNK, chunk_body, 0)

        for k in range(K):
            pltpu.sync_copy(out_v.at[k], out_hbm.at[k, pl.ds(base, SPT)])

    return sc_kernel


_SC_KERNEL = _build_kernel()


def _build_row_extract():
    CB = 65536
    grid = ((V + CB - 1) // CB,)
    in_spec = pl.BlockSpec((DIM, CB), lambda j: (0, j))
    out_spec = pl.BlockSpec((CB,), lambda j: (j,))

    def body(a_ref, b_ref, *out_refs):
        for d in range(DIM):
            out_refs[d][...] = a_ref[d, :]
            out_refs[DIM + d][...] = b_ref[d, :]

    return pl.pallas_call(
        body,
        grid=grid,
        in_specs=[in_spec, in_spec],
        out_specs=[out_spec] * (2 * DIM),
        out_shape=[jax.ShapeDtypeStruct((V,), jnp.float32)] * (2 * DIM),
    )


_ROW_EXTRACT = _build_row_extract()


@jax.jit
def kernel(input_index_batch, output_indices_batch, W_in, W_out):
    iidx = input_index_batch.astype(jnp.int32).reshape(B)
    oidx = output_indices_batch.astype(jnp.int32)
    oidx_cols = [oidx[:, k] for k in range(K)]
    rows = _ROW_EXTRACT(W_in, W_out.T)
    outT = _SC_KERNEL(iidx, *oidx_cols, *rows)
    return outT.T


# speedup vs baseline: 6.6275x; 1.1853x over previous
import functools

import jax
import jax.numpy as jnp
from jax import lax
from jax.experimental import pallas as pl
from jax.experimental.pallas import tpu as pltpu
from jax.experimental.pallas import tpu_sc as plsc

B = 16384
K = 21
DIM = 10
V = 1000000

NW = 32
SPT = B // NW
CS = 128
NCHUNK = SPT // CS


def _build_kernel():
    mesh = plsc.VectorSubcoreMesh(core_axis_name="c", subcore_axis_name="s")

    @functools.partial(
        pl.kernel,
        mesh=mesh,
        compiler_params=pltpu.CompilerParams(needs_layout_passes=False,
                                             use_tc_tiling_on_sc=False),
        out_type=jax.ShapeDtypeStruct((K, B), jnp.float32),
        scratch_types=[
            pltpu.VMEM((NCHUNK, CS), jnp.int32),
            pltpu.VMEM((DIM * NCHUNK, CS), jnp.float32),
            pltpu.VMEM((K * NCHUNK, CS), jnp.int32),
            pltpu.VMEM((2, DIM * K, CS), jnp.float32),
            pltpu.VMEM((K, SPT), jnp.float32),
            pltpu.SemaphoreType.DMA,
            pltpu.SemaphoreType.DMA,
            pltpu.SemaphoreType.DMA,
            pltpu.SemaphoreType.DMA,
        ],
    )
    def sc_kernel(*refs):
        idx_in_hbm = refs[0]
        oidx_refs = refs[1:1 + K]
        win_refs = refs[1 + K:1 + K + DIM]
        wout_refs = refs[1 + K + DIM:1 + K + 2 * DIM]
        out_hbm = refs[1 + K + 2 * DIM]
        (iidx_v, in_vals_v, oidx_v, vals_v, out_v,
         sem_st, sem_in, sem_a, sem_b) = refs[2 + K + 2 * DIM:]

        wid = lax.axis_index("c") * 16 + lax.axis_index("s")
        base = wid * SPT

        for p in range(NCHUNK):
            pltpu.async_copy(idx_in_hbm.at[pl.ds(base + p * CS, CS)],
                             iidx_v.at[p], sem_st)
        for k in range(K):
            def stage_o(c, carry, k=k):
                pltpu.async_copy(oidx_refs[k].at[pl.ds(base + c * CS, CS)],
                                 oidx_v.at[k * NCHUNK + c], sem_st)
                return carry

            lax.fori_loop(0, NCHUNK, stage_o, 0)
        for p in range(NCHUNK):
            pltpu.make_async_copy(idx_in_hbm.at[pl.ds(base + p * CS, CS)],
                                  iidx_v.at[p], sem_st).wait()
        for k in range(K):
            def stage_od(c, carry, k=k):
                pltpu.make_async_copy(oidx_refs[k].at[pl.ds(base + c * CS, CS)],
                                      oidx_v.at[k * NCHUNK + c], sem_st).wait()
                return carry

            lax.fori_loop(0, NCHUNK, stage_od, 0)

        for d in range(DIM):
            def fire_in(p, carry, d=d):
                pltpu.async_copy(win_refs[d].at[iidx_v.at[p]],
                                 in_vals_v.at[d * NCHUNK + p], sem_in)
                return carry

            lax.fori_loop(0, NCHUNK, fire_in, 0)

        def fire_out(c, buf, sem):
            for d in range(DIM):
                def fire1(k, carry, d=d):
                    pltpu.async_copy(wout_refs[d].at[oidx_v.at[k * NCHUNK + c]],
                                     vals_v.at[buf, d * K + k], sem)
                    return carry

                lax.fori_loop(0, K, fire1, 0)

        def drain_out(c, buf, sem):
            for d in range(DIM):
                def drain1(k, carry, d=d):
                    pltpu.make_async_copy(
                        wout_refs[d].at[oidx_v.at[k * NCHUNK + c]],
                        vals_v.at[buf, d * K + k], sem).wait()
                    return carry

                lax.fori_loop(0, K, drain1, 0)

        def compute(c, buf):
            def grp(g, carry):
                sbase = g * 16
                ivs = [in_vals_v[d * NCHUNK + c, pl.ds(sbase, 16)]
                       for d in range(DIM)]
                for k in range(K):
                    acc = vals_v[buf, k, pl.ds(sbase, 16)] * ivs[0]
                    for d in range(1, DIM):
                        acc = (acc +
                               vals_v[buf, d * K + k, pl.ds(sbase, 16)] * ivs[d])
                    out_v[k, pl.ds(c * CS + sbase, 16)] = acc
                return carry

            lax.fori_loop(0, CS // 16, grp, 0)

        fire_out(0, 0, sem_a)

        def drain_in():
            for d in range(DIM):
                def drain1(p, carry, d=d):
                    pltpu.make_async_copy(win_refs[d].at[iidx_v.at[p]],
                                          in_vals_v.at[d * NCHUNK + p],
                                          sem_in).wait()
                    return carry

                lax.fori_loop(0, NCHUNK, drain1, 0)

        drain_in()

        def pipe(cc, carry):
            c0 = cc * 2
            fire_out(c0 + 1, 1, sem_b)
            drain_out(c0, 0, sem_a)
            compute(c0, 0)

            @pl.when(cc == 0)
            def _():
                fire_out(c0 + 2, 0, sem_a)

            drain_out(c0 + 1, 1, sem_b)
            compute(c0 + 1, 1)
            return carry

        lax.fori_loop(0, NCHUNK // 2, pipe, 0)

        for k in range(K):
            pltpu.sync_copy(out_v.at[k], out_hbm.at[k, pl.ds(base, SPT)])

    return sc_kernel


_SC_KERNEL = _build_kernel()


def _build_row_extract():
    CB = 65536
    grid = ((V + CB - 1) // CB,)
    in_spec = pl.BlockSpec((DIM, CB), lambda j: (0, j))
    out_spec = pl.BlockSpec((CB,), lambda j: (j,))

    def body(a_ref, b_ref, *out_refs):
        for d in range(DIM):
            out_refs[d][...] = a_ref[d, :]
            out_refs[DIM + d][...] = b_ref[d, :]

    return pl.pallas_call(
        body,
        grid=grid,
        in_specs=[in_spec, in_spec],
        out_specs=[out_spec] * (2 * DIM),
        out_shape=[jax.ShapeDtypeStruct((V,), jnp.float32)] * (2 * DIM),
    )


_ROW_EXTRACT = _build_row_extract()


@jax.jit
def kernel(input_index_batch, output_indices_batch, W_in, W_out):
    iidx = input_index_batch.astype(jnp.int32).reshape(B)
    oidx = output_indices_batch.astype(jnp.int32)
    oidx_cols = [oidx[:, k] for k in range(K)]
    rows = _ROW_EXTRACT(W_in, W_out.T)
    outT = _SC_KERNEL(iidx, *oidx_cols, *rows)
    return outT.T
